# baseline (speedup 1.0000x reference)
import functools

import jax
import jax.numpy as jnp
from jax import lax
from jax.experimental import pallas as pl
from jax.experimental.pallas import tpu as pltpu
from jax.experimental.pallas import tpu_sc as plsc


def _sc_gather_rows(table, gidx):
    p, c = gidx.shape[0], table.shape[1]
    nw = 32
    b_per_w = p // nw
    mesh = plsc.VectorSubcoreMesh(core_axis_name="c", subcore_axis_name="s")

    @functools.partial(
        pl.kernel,
        mesh=mesh,
        out_type=jax.ShapeDtypeStruct((p, c), jnp.float32),
        scratch_types=[
            pltpu.VMEM((b_per_w,), jnp.int32),
            pltpu.VMEM((b_per_w, c), jnp.float32),
            pltpu.SemaphoreType.DMA,
        ],
    )
    def k(table_hbm, idx_hbm, out_hbm, idx_v, rows_v, sem):
        wid = lax.axis_index("s") * 2 + lax.axis_index("c")
        base = wid * b_per_w
        pltpu.sync_copy(idx_hbm.at[pl.ds(base, b_per_w)], idx_v)
        pltpu.async_copy(table_hbm.at[idx_v], rows_v, sem).wait()
        pltpu.sync_copy(rows_v, out_hbm.at[pl.ds(base, b_per_w)])

    return k(table, gidx)


def _nn_match_kernel(s_ref, t_ref, idx_ref):
    s = s_ref[0]
    t = t_ref[0]
    st = jax.lax.dot_general(
        s.astype(jnp.bfloat16), t.astype(jnp.bfloat16),
        (((0,), (1,)), ((), ())), preferred_element_type=jnp.float32
    )
    s2 = jnp.sum(s * s, axis=0, keepdims=True)
    t2 = jnp.sum(t * t, axis=1, keepdims=True)
    d2 = (s2.T + t2.T) - 2.0 * st
    idx_ref[0, 0] = jnp.argmin(d2, axis=1)


def _nn_concat(src, tar):
    b, c, h, w = src.shape
    n = h * w
    s = src.reshape(b, c, n)
    t = tar.reshape(b, c, n).transpose(0, 2, 1)
    idx = pl.pallas_call(
        _nn_match_kernel,
        grid=(b,),
        in_specs=[
            pl.BlockSpec((1, c, n), lambda i: (i, 0, 0)),
            pl.BlockSpec((1, n, c), lambda i: (i, 0, 0)),
        ],
        out_specs=pl.BlockSpec((1, 1, n), lambda i: (i, 0, 0)),
        out_shape=jax.ShapeDtypeStruct((b, 1, n), jnp.int32),
    )(s, t)
    idx = idx.reshape(b, n)
    gidx = (idx + n * jnp.arange(b, dtype=jnp.int32)[:, None]).reshape(b * n)
    p = -(-(b * n) // 256) * 256
    gidx = jnp.pad(gidx, (0, p - b * n))
    nearest = _sc_gather_rows(t.reshape(b * n, c), gidx)
    nearest = nearest[:b * n].reshape(b, n, c)
    nearest = nearest.transpose(0, 2, 1).reshape(b, c, h, w)
    return jnp.concatenate([src, nearest], axis=1)


def _conv2d(x, w, b):
    y = jax.lax.conv_general_dilated(
        x, w, (1, 1), 'VALID', dimension_numbers=('NCHW', 'OIHW', 'NCHW')
    )
    return y + b[None, :, None, None]


def _conv_t2(x, w, b):
    y = jax.lax.conv_transpose(
        x, w, (2, 2), 'VALID', dimension_numbers=('NCHW', 'OIHW', 'NCHW')
    )
    return y + b[None, :, None, None]


def _bn(x, g, be):
    m = jnp.mean(x, axis=(0, 2, 3), keepdims=True)
    v = jnp.var(x, axis=(0, 2, 3), keepdims=True)
    return (x - m) / jnp.sqrt(v + 1e-5) * g[None, :, None, None] + be[None, :, None, None]


def _maxpool2(x):
    return jax.lax.reduce_window(
        x, -jnp.inf, jax.lax.max, (1, 1, 2, 2), (1, 1, 2, 2), 'VALID'
    )


def _enc_block(x, p):
    x = jax.nn.relu(_bn(_conv2d(x, p['w1'], p['b1']), p['g1'], p['be1']))
    x = jax.nn.relu(_bn(_conv2d(x, p['w2'], p['b2']), p['g2'], p['be2']))
    return _maxpool2(x)


def _dec_block(x, p):
    x = jax.nn.relu(_bn(_conv2d(x, p['w1'], p['b1']), p['g1'], p['be1']))
    x = jax.nn.relu(_bn(_conv2d(x, p['w2'], p['b2']), p['g2'], p['be2']))
    return _conv_t2(x, p['wt'], p['bt'])


def _bn2(x, g, be):
    b2, c, h, w = x.shape
    xg = x.reshape(2, b2 // 2, c, h, w)
    m = jnp.mean(xg, axis=(1, 3, 4), keepdims=True)
    v = jnp.var(xg, axis=(1, 3, 4), keepdims=True)
    y = (xg - m) / jnp.sqrt(v + 1e-5) * g[None, None, :, None, None] \
        + be[None, None, :, None, None]
    return y.reshape(b2, c, h, w)


def _enc_block2(x, p):
    x = jax.nn.relu(_bn2(_conv2d(x, p['w1'], p['b1']), p['g1'], p['be1']))
    x = jax.nn.relu(_bn2(_conv2d(x, p['w2'], p['b2']), p['g2'], p['be2']))
    return _maxpool2(x)


def kernel(src_img, tar_img, params):
    s1 = _enc_block(src_img, params['enc1'])
    s2 = _enc_block(s1, params['enc2'])
    s3 = _enc_block(s2, params['enc3'])
    s4 = _enc_block(s3, params['enc4'])
    t1 = _enc_block(tar_img, params['enc1'])
    t2 = _enc_block(t1, params['enc2'])
    t3 = _enc_block(t2, params['enc3'])
    t4 = _enc_block(t3, params['enc4'])
    c3 = _nn_concat(s3, t3)
    c4 = _nn_concat(s4, t4)
    c4u = jax.image.resize(
        c4, (c4.shape[0], c4.shape[1], c3.shape[2], c3.shape[3]), method='bilinear'
    )
    d = _dec_block(jnp.concatenate([c3, c4u], axis=1), params['dec3'])
    d = _dec_block(d, params['dec2'])
    d = _conv2d(d, params['dec1']['w'], params['dec1']['b'])
    pred = jax.image.resize(
        d, (d.shape[0], d.shape[1], src_img.shape[2], src_img.shape[3]),
        method='bilinear',
    )
    return pred


# speedup vs baseline: 1.0046x; 1.0046x over previous
import functools

import jax
import jax.numpy as jnp
from jax import lax
from jax.experimental import pallas as pl
from jax.experimental.pallas import tpu as pltpu
from jax.experimental.pallas import tpu_sc as plsc


def _sc_gather_rows(table, gidx):
    p, c = gidx.shape[0], table.shape[1]
    nw = 32
    b_per_w = p // nw
    mesh = plsc.VectorSubcoreMesh(core_axis_name="c", subcore_axis_name="s")

    @functools.partial(
        pl.kernel,
        mesh=mesh,
        out_type=jax.ShapeDtypeStruct((p, c), jnp.float32),
        scratch_types=[
            pltpu.VMEM((b_per_w,), jnp.int32),
            pltpu.VMEM((b_per_w, c), jnp.float32),
            pltpu.SemaphoreType.DMA,
        ],
    )
    def k(table_hbm, idx_hbm, out_hbm, idx_v, rows_v, sem):
        wid = lax.axis_index("s") * 2 + lax.axis_index("c")
        base = wid * b_per_w
        pltpu.sync_copy(idx_hbm.at[pl.ds(base, b_per_w)], idx_v)
        pltpu.async_copy(table_hbm.at[idx_v], rows_v, sem).wait()
        pltpu.sync_copy(rows_v, out_hbm.at[pl.ds(base, b_per_w)])

    return k(table, gidx)


def _nn_match_kernel(s_ref, t_ref, idx_ref):
    s = s_ref[0]
    t = t_ref[0]
    st = jax.lax.dot_general(
        s.astype(jnp.bfloat16), t.astype(jnp.bfloat16),
        (((1,), (1,)), ((), ())), preferred_element_type=jnp.float32
    )
    s2 = jnp.sum(s * s, axis=1, keepdims=True)
    t2 = jnp.sum(t * t, axis=1, keepdims=True)
    d2 = (s2 + t2.T) - 2.0 * st
    idx_ref[0, 0] = jnp.argmin(d2, axis=1)


def _nn_concat(src, tar):
    b, c, h, w = src.shape
    n = h * w
    s = src.reshape(b, c, n).transpose(0, 2, 1)
    t = tar.reshape(b, c, n).transpose(0, 2, 1)
    idx = pl.pallas_call(
        _nn_match_kernel,
        grid=(b,),
        in_specs=[
            pl.BlockSpec((1, n, c), lambda i: (i, 0, 0)),
            pl.BlockSpec((1, n, c), lambda i: (i, 0, 0)),
        ],
        out_specs=pl.BlockSpec((1, 1, n), lambda i: (i, 0, 0)),
        out_shape=jax.ShapeDtypeStruct((b, 1, n), jnp.int32),
    )(s, t)
    idx = idx.reshape(b, n)
    gidx = (idx + n * jnp.arange(b, dtype=jnp.int32)[:, None]).reshape(b * n)
    p = -(-(b * n) // 256) * 256
    gidx = jnp.pad(gidx, (0, p - b * n))
    nearest = _sc_gather_rows(t.reshape(b * n, c), gidx)
    nearest = nearest[:b * n].reshape(b, n, c)
    nearest = nearest.transpose(0, 2, 1).reshape(b, c, h, w)
    return jnp.concatenate([src, nearest], axis=1)


def _conv2d(x, w, b):
    y = jax.lax.conv_general_dilated(
        x, w, (1, 1), 'VALID', dimension_numbers=('NCHW', 'OIHW', 'NCHW')
    )
    return y + b[None, :, None, None]


def _conv_t2(x, w, b):
    y = jax.lax.conv_transpose(
        x, w, (2, 2), 'VALID', dimension_numbers=('NCHW', 'OIHW', 'NCHW')
    )
    return y + b[None, :, None, None]


def _bn(x, g, be):
    m = jnp.mean(x, axis=(0, 2, 3), keepdims=True)
    v = jnp.var(x, axis=(0, 2, 3), keepdims=True)
    return (x - m) / jnp.sqrt(v + 1e-5) * g[None, :, None, None] + be[None, :, None, None]


def _maxpool2(x):
    return jax.lax.reduce_window(
        x, -jnp.inf, jax.lax.max, (1, 1, 2, 2), (1, 1, 2, 2), 'VALID'
    )


def _enc_block(x, p):
    x = jax.nn.relu(_bn(_conv2d(x, p['w1'], p['b1']), p['g1'], p['be1']))
    x = jax.nn.relu(_bn(_conv2d(x, p['w2'], p['b2']), p['g2'], p['be2']))
    return _maxpool2(x)


def _dec_block(x, p):
    x = jax.nn.relu(_bn(_conv2d(x, p['w1'], p['b1']), p['g1'], p['be1']))
    x = jax.nn.relu(_bn(_conv2d(x, p['w2'], p['b2']), p['g2'], p['be2']))
    return _conv_t2(x, p['wt'], p['bt'])


def _bn2(x, g, be):
    b2, c, h, w = x.shape
    xg = x.reshape(2, b2 // 2, c, h, w)
    m = jnp.mean(xg, axis=(1, 3, 4), keepdims=True)
    v = jnp.var(xg, axis=(1, 3, 4), keepdims=True)
    y = (xg - m) / jnp.sqrt(v + 1e-5) * g[None, None, :, None, None] \
        + be[None, None, :, None, None]
    return y.reshape(b2, c, h, w)


def _enc_block2(x, p):
    x = jax.nn.relu(_bn2(_conv2d(x, p['w1'], p['b1']), p['g1'], p['be1']))
    x = jax.nn.relu(_bn2(_conv2d(x, p['w2'], p['b2']), p['g2'], p['be2']))
    return _maxpool2(x)


def kernel(src_img, tar_img, params):
    s1 = _enc_block(src_img, params['enc1'])
    s2 = _enc_block(s1, params['enc2'])
    s3 = _enc_block(s2, params['enc3'])
    s4 = _enc_block(s3, params['enc4'])
    t1 = _enc_block(tar_img, params['enc1'])
    t2 = _enc_block(t1, params['enc2'])
    t3 = _enc_block(t2, params['enc3'])
    t4 = _enc_block(t3, params['enc4'])
    c3 = _nn_concat(s3, t3)
    c4 = _nn_concat(s4, t4)
    c4u = jax.image.resize(
        c4, (c4.shape[0], c4.shape[1], c3.shape[2], c3.shape[3]), method='bilinear'
    )
    d = _dec_block(jnp.concatenate([c3, c4u], axis=1), params['dec3'])
    d = _dec_block(d, params['dec2'])
    d = _conv2d(d, params['dec1']['w'], params['dec1']['b'])
    pred = jax.image.resize(
        d, (d.shape[0], d.shape[1], src_img.shape[2], src_img.shape[3]),
        method='bilinear',
    )
    return pred


# speedup vs baseline: 1.0054x; 1.0009x over previous
import functools

import jax
import jax.numpy as jnp
from jax import lax
from jax.experimental import pallas as pl
from jax.experimental.pallas import tpu as pltpu
from jax.experimental.pallas import tpu_sc as plsc


def _sc_gather_rows(table, gidx):
    p, c = gidx.shape[0], table.shape[1]
    nw = 32
    b_per_w = p // nw
    mesh = plsc.VectorSubcoreMesh(core_axis_name="c", subcore_axis_name="s")

    @functools.partial(
        pl.kernel,
        mesh=mesh,
        out_type=jax.ShapeDtypeStruct((p, c), jnp.float32),
        scratch_types=[
            pltpu.VMEM((b_per_w,), jnp.int32),
            pltpu.VMEM((b_per_w, c), jnp.float32),
            pltpu.SemaphoreType.DMA,
        ],
    )
    def k(table_hbm, idx_hbm, out_hbm, idx_v, rows_v, sem):
        wid = lax.axis_index("s") * 2 + lax.axis_index("c")
        base = wid * b_per_w
        pltpu.sync_copy(idx_hbm.at[pl.ds(base, b_per_w)], idx_v)
        pltpu.async_copy(table_hbm.at[idx_v], rows_v, sem).wait()
        pltpu.sync_copy(rows_v, out_hbm.at[pl.ds(base, b_per_w)])

    return k(table, gidx)


def _match_one(s, t):
    st = jax.lax.dot_general(
        s.astype(jnp.bfloat16), t.astype(jnp.bfloat16),
        (((1,), (1,)), ((), ())), preferred_element_type=jnp.float32
    )
    s2 = jnp.sum(s * s, axis=1, keepdims=True)
    t2 = jnp.sum(t * t, axis=1, keepdims=True)
    d2 = (s2 + t2.T) - 2.0 * st
    return jnp.argmin(d2, axis=1)


def _nn_match_kernel(s3_ref, t3_ref, s4_ref, t4_ref, idx3_ref, idx4_ref):
    idx3_ref[0, 0] = _match_one(s3_ref[0], t3_ref[0])
    idx4_ref[0, 0] = _match_one(s4_ref[0], t4_ref[0])


def _nn_concat_pair(s3f, t3f, s4f, t4f):
    b, c3, h3, w3 = s3f.shape
    _, c4, h4, w4 = s4f.shape
    n3, n4 = h3 * w3, h4 * w4
    s3 = s3f.reshape(b, c3, n3).transpose(0, 2, 1)
    t3 = t3f.reshape(b, c3, n3).transpose(0, 2, 1)
    s4 = s4f.reshape(b, c4, n4).transpose(0, 2, 1)
    t4 = t4f.reshape(b, c4, n4).transpose(0, 2, 1)
    idx3, idx4 = pl.pallas_call(
        _nn_match_kernel,
        grid=(b,),
        in_specs=[
            pl.BlockSpec((1, n3, c3), lambda i: (i, 0, 0)),
            pl.BlockSpec((1, n3, c3), lambda i: (i, 0, 0)),
            pl.BlockSpec((1, n4, c4), lambda i: (i, 0, 0)),
            pl.BlockSpec((1, n4, c4), lambda i: (i, 0, 0)),
        ],
        out_specs=[
            pl.BlockSpec((1, 1, n3), lambda i: (i, 0, 0)),
            pl.BlockSpec((1, 1, n4), lambda i: (i, 0, 0)),
        ],
        out_shape=[
            jax.ShapeDtypeStruct((b, 1, n3), jnp.int32),
            jax.ShapeDtypeStruct((b, 1, n4), jnp.int32),
        ],
    )(s3, t3, s4, t4)
    assert c4 == 2 * c3
    r3 = b * n3
    table = jnp.concatenate(
        [t3.reshape(r3, c3), t4.reshape(b * n4 * 2, c3)], axis=0
    )
    boff3 = n3 * jnp.arange(b, dtype=jnp.int32)[:, None]
    boff4 = n4 * jnp.arange(b, dtype=jnp.int32)[:, None]
    g3 = (idx3.reshape(b, n3) + boff3).reshape(r3)
    g4 = (idx4.reshape(b, n4) + boff4).reshape(b * n4)
    g4 = r3 + 2 * g4
    g4 = jnp.stack([g4, g4 + 1], axis=-1).reshape(2 * b * n4)
    total = r3 + 2 * b * n4
    p = -(-total // 256) * 256
    gidx = jnp.pad(jnp.concatenate([g3, g4]), (0, p - total))
    rows = _sc_gather_rows(table, gidx)
    near3 = rows[:r3].reshape(b, n3, c3).transpose(0, 2, 1).reshape(b, c3, h3, w3)
    near4 = rows[r3:total].reshape(b, n4, c4).transpose(0, 2, 1).reshape(b, c4, h4, w4)
    return (jnp.concatenate([s3f, near3], axis=1),
            jnp.concatenate([s4f, near4], axis=1))


def _conv2d(x, w, b):
    y = jax.lax.conv_general_dilated(
        x, w, (1, 1), 'VALID', dimension_numbers=('NCHW', 'OIHW', 'NCHW')
    )
    return y + b[None, :, None, None]


def _conv_t2(x, w, b):
    y = jax.lax.conv_transpose(
        x, w, (2, 2), 'VALID', dimension_numbers=('NCHW', 'OIHW', 'NCHW')
    )
    return y + b[None, :, None, None]


def _bn(x, g, be):
    m = jnp.mean(x, axis=(0, 2, 3), keepdims=True)
    v = jnp.var(x, axis=(0, 2, 3), keepdims=True)
    return (x - m) / jnp.sqrt(v + 1e-5) * g[None, :, None, None] + be[None, :, None, None]


def _maxpool2(x):
    return jax.lax.reduce_window(
        x, -jnp.inf, jax.lax.max, (1, 1, 2, 2), (1, 1, 2, 2), 'VALID'
    )


def _enc_block(x, p):
    x = jax.nn.relu(_bn(_conv2d(x, p['w1'], p['b1']), p['g1'], p['be1']))
    x = jax.nn.relu(_bn(_conv2d(x, p['w2'], p['b2']), p['g2'], p['be2']))
    return _maxpool2(x)


def _dec_block(x, p):
    x = jax.nn.relu(_bn(_conv2d(x, p['w1'], p['b1']), p['g1'], p['be1']))
    x = jax.nn.relu(_bn(_conv2d(x, p['w2'], p['b2']), p['g2'], p['be2']))
    return _conv_t2(x, p['wt'], p['bt'])


def _bn2(x, g, be):
    b2, c, h, w = x.shape
    xg = x.reshape(2, b2 // 2, c, h, w)
    m = jnp.mean(xg, axis=(1, 3, 4), keepdims=True)
    v = jnp.var(xg, axis=(1, 3, 4), keepdims=True)
    y = (xg - m) / jnp.sqrt(v + 1e-5) * g[None, None, :, None, None] \
        + be[None, None, :, None, None]
    return y.reshape(b2, c, h, w)


def _enc_block2(x, p):
    x = jax.nn.relu(_bn2(_conv2d(x, p['w1'], p['b1']), p['g1'], p['be1']))
    x = jax.nn.relu(_bn2(_conv2d(x, p['w2'], p['b2']), p['g2'], p['be2']))
    return _maxpool2(x)


def kernel(src_img, tar_img, params):
    s1 = _enc_block(src_img, params['enc1'])
    s2 = _enc_block(s1, params['enc2'])
    s3 = _enc_block(s2, params['enc3'])
    s4 = _enc_block(s3, params['enc4'])
    t1 = _enc_block(tar_img, params['enc1'])
    t2 = _enc_block(t1, params['enc2'])
    t3 = _enc_block(t2, params['enc3'])
    t4 = _enc_block(t3, params['enc4'])
    c3, c4 = _nn_concat_pair(s3, t3, s4, t4)
    c4u = jax.image.resize(
        c4, (c4.shape[0], c4.shape[1], c3.shape[2], c3.shape[3]), method='bilinear'
    )
    d = _dec_block(jnp.concatenate([c3, c4u], axis=1), params['dec3'])
    d = _dec_block(d, params['dec2'])
    d = _conv2d(d, params['dec1']['w'], params['dec1']['b'])
    pred = jax.image.resize(
        d, (d.shape[0], d.shape[1], src_img.shape[2], src_img.shape[3]),
        method='bilinear',
    )
    return pred
